# baseline (speedup 1.0000x reference)
import functools

import jax
import jax.numpy as jnp
from jax import lax
from jax.experimental import pallas as pl
from jax.experimental.pallas import tpu as pltpu
from jax.experimental.pallas import tpu_sc as plsc

N = 16384
E = 262144
D = 256
DE = 16
B = 256
L = 128


_EB = 2048
_NB = 1024


def _z_body(ea_ref, w0_ref, w1_ref, w2_ref, b0_ref, b1_ref, b2_ref,
            z0_ref, z1_ref, z2_ref):
    ea = ea_ref[...]
    z0_ref[...] = jnp.dot(ea, w0_ref[...], preferred_element_type=jnp.float32) + b0_ref[...]
    z1_ref[...] = jnp.dot(ea, w1_ref[...], preferred_element_type=jnp.float32) + b1_ref[...]
    z2_ref[...] = jnp.dot(ea, w2_ref[...], preferred_element_type=jnp.float32) + b2_ref[...]


def _z_call(ea, w0, w1, w2, b0, b1, b2):
    zspec = pl.BlockSpec((_EB, D), lambda i: (i, 0))
    wspec = pl.BlockSpec((DE, D), lambda i: (0, 0))
    bspec = pl.BlockSpec((1, D), lambda i: (0, 0))
    return pl.pallas_call(
        _z_body,
        grid=(E // _EB,),
        in_specs=[pl.BlockSpec((_EB, DE), lambda i: (i, 0)),
                  wspec, wspec, wspec, bspec, bspec, bspec],
        out_specs=[zspec, zspec, zspec],
        out_shape=[jax.ShapeDtypeStruct((E, D), jnp.float32)] * 3,
    )(ea, w0, w1, w2, b0, b1, b2)


def _pre_body(h_ref, w1_ref, ws_ref, y_ref, s_ref):
    h = h_ref[...]
    y_ref[...] = jnp.dot(h, w1_ref[...], preferred_element_type=jnp.float32)
    s_ref[...] = jnp.dot(h, ws_ref[...], preferred_element_type=jnp.float32)


def _mid_body(agg_ref, sp_ref, w1_ref, ws_ref, y_ref, s_ref):
    h = jnp.maximum(agg_ref[...] + sp_ref[...], 0.0)
    y_ref[...] = jnp.dot(h, w1_ref[...], preferred_element_type=jnp.float32)
    s_ref[...] = jnp.dot(h, ws_ref[...], preferred_element_type=jnp.float32)


def _h_specs():
    nspec = pl.BlockSpec((_NB, D), lambda i: (i, 0))
    wspec = pl.BlockSpec((D, D), lambda i: (0, 0))
    return nspec, wspec


def _pre_call(h, w1, ws):
    nspec, wspec = _h_specs()
    return pl.pallas_call(
        _pre_body,
        grid=(N // _NB,),
        in_specs=[nspec, wspec, wspec],
        out_specs=[nspec, nspec],
        out_shape=[jax.ShapeDtypeStruct((N, D), jnp.float32)] * 2,
    )(h, w1, ws)


def _mid_call(agg, sp, w1, ws):
    nspec, wspec = _h_specs()
    return pl.pallas_call(
        _mid_body,
        grid=(N // _NB,),
        in_specs=[nspec, nspec, wspec, wspec],
        out_specs=[nspec, nspec],
        out_shape=[jax.ShapeDtypeStruct((N, D), jnp.float32)] * 2,
    )(agg, sp, w1, ws)


_NP = N + 2 * L


def _fin_body(agg_ref, sp_ref, o_ref):
    i = pl.program_id(0)
    h = jnp.maximum(agg_ref[...] + sp_ref[...], 0.0)
    row = i * 128 + lax.broadcasted_iota(jnp.int32, (128, 1), 0)
    o_ref[...] = jnp.where(row < N, h, 0.0)


def _fin_call(agg, sp):
    nspec = pl.BlockSpec((128, D), lambda i: (jnp.minimum(i, 127), 0))
    return pl.pallas_call(
        _fin_body,
        grid=(_NP // 128,),
        in_specs=[nspec, nspec],
        out_specs=pl.BlockSpec((128, D), lambda i: (i, 0)),
        out_shape=jax.ShapeDtypeStruct((_NP, D), jnp.float32),
    )(agg, sp)


def _seq_body(bi_ref, h3_ref, o_ref):
    b = pl.program_id(0)
    bi = bi_ref[...]
    first = jnp.sum((bi < b).astype(jnp.int32))
    cnt = jnp.sum((bi == b).astype(jnp.int32))
    base = pl.multiple_of((first // 8) * 8, 8)
    rem = first - base
    window = h3_ref[pl.ds(base, L + 8), :]
    rows = pltpu.roll(window, (L + 8) - rem, 0)[:L]
    liota = lax.broadcasted_iota(jnp.int32, (L, 1), 0)
    o_ref[0] = jnp.where(liota < cnt, rows, 0.0)


def _seq_call(bi2d, h3p):
    return pl.pallas_call(
        _seq_body,
        grid=(B,),
        in_specs=[pl.BlockSpec((128, 128), lambda b: (0, 0)),
                  pl.BlockSpec((_NP, D), lambda b: (0, 0))],
        out_specs=pl.BlockSpec((1, L, D), lambda b: (b, 0, 0)),
        out_shape=jax.ShapeDtypeStruct((B, L, D), jnp.float32),
    )(bi2d, h3p)


_NW = 32
_OWN = N // 64
_S = 8192
_G = 48
_W = 256
_NBLK = E // _S
_CAP = E + 16 * _NBLK
_RC = 2048

_mesh = plsc.VectorSubcoreMesh(core_axis_name="c", subcore_axis_name="s")


def _prep_body(src_hbm, dst_hbm, rsrc_hbm, reid_hbm, rdl_hbm, cnts_hbm,
               dstb, srcb, cw_s, cw_e, cw_d, cbuf, sem):
    c = lax.axis_index("c")
    s = lax.axis_index("s")
    w = s * 2 + c
    zero16i = jnp.zeros((16,), jnp.int32)
    iota16 = lax.iota(jnp.int32, 16)
    sent16 = jnp.full((16,), _OWN, jnp.int32)

    def zstage(i, _):
        cw_s[pl.ds(i * 16, 16)] = zero16i
        cw_e[pl.ds(i * 16, 16)] = zero16i
        cw_d[pl.ds(i * 16, 16)] = zero16i
        return 0
    lax.fori_loop(0, (_S + _W + 16) // 16, zstage, 0)

    for p in range(2):
        bucket = p * _NW + w
        own_base = bucket * _OWN
        rbase = bucket * _CAP

        def block_body(bi, cur):
            base = bi * _S
            pltpu.sync_copy(dst_hbm.at[pl.ds(base, _S)], dstb)
            pltpu.sync_copy(src_hbm.at[pl.ds(base, _S)], srcb)

            def scan(i, st):
                dv = dstb[pl.ds(i * 16, 16)]
                m = (dv >= own_base) & (dv < own_base + _OWN)
                sv = srcb[pl.ds(i * 16, 16)]
                ev = iota16 + (base + i * 16)
                pos = st + plsc.cumsum(m.astype(jnp.int32)) - 1
                plsc.store_scatter(cw_d, [pos], dv - own_base, mask=m)
                plsc.store_scatter(cw_s, [pos], sv, mask=m)
                plsc.store_scatter(cw_e, [pos], ev, mask=m)
                cnt = plsc.all_reduce_population_count(m)
                return st + cnt[0]

            n = lax.fori_loop(0, _S // 16, scan, 0)
            plsc.store_scatter(cw_d, [n + iota16], sent16)
            plsc.store_scatter(cw_s, [n + iota16], zero16i)
            plsc.store_scatter(cw_e, [n + iota16], zero16i)
            np_ = ((n + 15) // 16) * 16
            nw = (np_ + _W - 1) // _W

            def flush(k, _):
                o = pl.multiple_of(rbase + cur + k * _W, 16)
                pltpu.sync_copy(cw_s.at[pl.ds(k * _W, _W)],
                                rsrc_hbm.at[pl.ds(o, _W)])
                pltpu.sync_copy(cw_e.at[pl.ds(k * _W, _W)],
                                reid_hbm.at[pl.ds(o, _W)])
                pltpu.sync_copy(cw_d.at[pl.ds(k * _W, _W)],
                                rdl_hbm.at[pl.ds(o, _W)])
                return 0

            lax.fori_loop(0, nw, flush, 0)
            return cur + np_

        total = lax.fori_loop(0, _NBLK, block_body, 0)
        cbuf[pl.ds(0, 16)] = jnp.full((16,), total, jnp.int32)
        pltpu.sync_copy(cbuf, cnts_hbm.at[pl.ds(bucket * 16, 16)])


_sc_prep = functools.partial(
    pl.kernel,
    mesh=_mesh,
    compiler_params=pltpu.CompilerParams(needs_layout_passes=False),
    out_type=[
        jax.ShapeDtypeStruct((64 * _CAP,), jnp.int32),
        jax.ShapeDtypeStruct((64 * _CAP,), jnp.int32),
        jax.ShapeDtypeStruct((64 * _CAP,), jnp.int32),
        jax.ShapeDtypeStruct((64 * 16,), jnp.int32),
    ],
    scratch_types=[
        pltpu.VMEM((_S,), jnp.int32),
        pltpu.VMEM((_S,), jnp.int32),
        pltpu.VMEM((_S + _W + 16,), jnp.int32),
        pltpu.VMEM((_S + _W + 16,), jnp.int32),
        pltpu.VMEM((_S + _W + 16,), jnp.int32),
        pltpu.VMEM((16,), jnp.int32),
        pltpu.SemaphoreType.DMA,
    ],
)(_prep_body)


def _layer_body(y_hbm, z_hbm, rsrc_hbm, reid_hbm, rdl_hbm, cnts_hbm, agg_hbm,
                rc_s, rc_e, rc_d, cbuf,
                wrow0, wrow1, yrow0, yrow1, acc,
                semz0, semz1, semy0, semy1):
    c = lax.axis_index("c")
    s = lax.axis_index("s")
    w = s * 2 + c
    zero16f = jnp.zeros((16,), jnp.float32)
    zero16i = jnp.zeros((16,), jnp.int32)

    def zrc(i, _):
        rc_s[pl.ds(i * 16, 16)] = zero16i
        rc_e[pl.ds(i * 16, 16)] = zero16i
        rc_d[pl.ds(i * 16, 16)] = zero16i
        return 0
    lax.fori_loop(0, _RC // 16, zrc, 0)

    def zero_acc(i, _):
        for j in range(D // 16):
            acc[i, pl.ds(j * 16, 16)] = zero16f
        return 0

    def issue(goff, wrow, yrow, semz, semy):
        pltpu.async_copy(z_hbm.at[rc_e.at[pl.ds(goff, _G)]], wrow, semz)
        pltpu.async_copy(y_hbm.at[rc_s.at[pl.ds(goff, _G)]], yrow, semy)

    def drain(wrow, yrow, semz, semy):
        pltpu.make_async_copy(z_hbm.at[pl.ds(0, _G)], wrow, semz).wait()
        pltpu.make_async_copy(y_hbm.at[pl.ds(0, _G)], yrow, semy).wait()

    for p in range(2):
        bucket = p * _NW + w
        own_base = bucket * _OWN
        rbase = bucket * _CAP
        pltpu.sync_copy(cnts_hbm.at[pl.ds(bucket * 16, 16)], cbuf)
        cnt = cbuf[pl.ds(0, 16)][0]
        lax.fori_loop(0, _OWN + 1, zero_acc, 0)

        nchunk = (cnt + _RC - 1) // _RC

        def chunk_body(ci, _):
            c0 = ci * _RC
            n_in = jnp.minimum(_RC, cnt - c0)
            f0 = pl.multiple_of(rbase + c0, 16)
            pltpu.sync_copy(rsrc_hbm.at[pl.ds(f0, _RC)], rc_s)
            pltpu.sync_copy(reid_hbm.at[pl.ds(f0, _RC)], rc_e)
            pltpu.sync_copy(rdl_hbm.at[pl.ds(f0, _RC)], rc_d)
            ngrp = (n_in + _G - 1) // _G

            def compute(goff, wrow, yrow):
                nsub = (jnp.minimum(_G, n_in - goff) + 15) // 16

                def subfn(h, _):
                    o16 = h * 16
                    dv = rc_d[pl.ds(goff + o16, 16)]
                    for i in range(16):
                        dloc = dv[i]
                        for j in range(D // 16):
                            v = (wrow[o16 + i, pl.ds(j * 16, 16)]
                                 + yrow[o16 + i, pl.ds(j * 16, 16)])
                            plsc.addupdate(acc.at[dloc, pl.ds(j * 16, 16)],
                                           jnp.maximum(v, 0.0))
                    return 0

                lax.fori_loop(0, nsub, subfn, 0)

            @pl.when(ngrp > 0)
            def _():
                issue(0, wrow0, yrow0, semz0, semy0)

            def pair(gg, _):
                g0 = 2 * gg
                g1 = g0 + 1

                @pl.when(g1 < ngrp)
                def _():
                    issue(g1 * _G, wrow1, yrow1, semz1, semy1)

                drain(wrow0, yrow0, semz0, semy0)
                compute(g0 * _G, wrow0, yrow0)

                @pl.when(g1 < ngrp)
                def _():
                    @pl.when(g1 + 1 < ngrp)
                    def _():
                        issue((g1 + 1) * _G, wrow0, yrow0, semz0, semy0)

                    drain(wrow1, yrow1, semz1, semy1)
                    compute(g1 * _G, wrow1, yrow1)

                return 0

            lax.fori_loop(0, (ngrp + 1) // 2, pair, 0)
            return 0

        lax.fori_loop(0, nchunk, chunk_body, 0)
        pltpu.sync_copy(acc.at[pl.ds(0, _OWN)],
                        agg_hbm.at[pl.ds(own_base, _OWN)])


_sc_layer = functools.partial(
    pl.kernel,
    mesh=_mesh,
    compiler_params=pltpu.CompilerParams(needs_layout_passes=False),
    out_type=jax.ShapeDtypeStruct((N, D), jnp.float32),
    scratch_types=[
        pltpu.VMEM((_RC,), jnp.int32),
        pltpu.VMEM((_RC,), jnp.int32),
        pltpu.VMEM((_RC,), jnp.int32),
        pltpu.VMEM((16,), jnp.int32),
        pltpu.VMEM((_G, D), jnp.float32),
        pltpu.VMEM((_G, D), jnp.float32),
        pltpu.VMEM((_G, D), jnp.float32),
        pltpu.VMEM((_G, D), jnp.float32),
        pltpu.VMEM((_OWN + 1, D), jnp.float32),
        pltpu.SemaphoreType.DMA,
        pltpu.SemaphoreType.DMA,
        pltpu.SemaphoreType.DMA,
        pltpu.SemaphoreType.DMA,
    ],
)(_layer_body)


def kernel(x, edge_index, edge_attr, pos, batch_indices,
           W1_0, We_0, Ws_0, b_0,
           W1_1, We_1, Ws_1, b_1,
           W1_2, We_2, Ws_2, b_2):
    src = edge_index[0].astype(jnp.int32)
    dst = edge_index[1].astype(jnp.int32)
    bi2d = batch_indices.astype(jnp.int32).reshape(128, 128)

    rsrc, reid, rdl, cnts = _sc_prep(src, dst)
    z0, z1, z2 = _z_call(edge_attr, We_0, We_1, We_2,
                         b_0.reshape(1, D), b_1.reshape(1, D), b_2.reshape(1, D))
    y, sp = _pre_call(x, W1_0, Ws_0)
    agg = _sc_layer(y, z0, rsrc, reid, rdl, cnts)
    y, sp = _mid_call(agg, sp, W1_1, Ws_1)
    agg = _sc_layer(y, z1, rsrc, reid, rdl, cnts)
    y, sp = _mid_call(agg, sp, W1_2, Ws_2)
    agg = _sc_layer(y, z2, rsrc, reid, rdl, cnts)
    h3p = _fin_call(agg, sp)
    return _seq_call(bi2d, h3p)


# speedup vs baseline: 1.3939x; 1.3939x over previous
import functools

import jax
import jax.numpy as jnp
from jax import lax
from jax.experimental import pallas as pl
from jax.experimental.pallas import tpu as pltpu
from jax.experimental.pallas import tpu_sc as plsc

N = 16384
E = 262144
D = 256
DE = 16
B = 256
L = 128


_EB = 2048
_NB = 1024


def _z_body(ea_ref, w0_ref, w1_ref, w2_ref, b0_ref, b1_ref, b2_ref,
            z0_ref, z1_ref, z2_ref):
    ea = ea_ref[...]
    z0_ref[...] = jnp.dot(ea, w0_ref[...], preferred_element_type=jnp.float32) + b0_ref[...]
    z1_ref[...] = jnp.dot(ea, w1_ref[...], preferred_element_type=jnp.float32) + b1_ref[...]
    z2_ref[...] = jnp.dot(ea, w2_ref[...], preferred_element_type=jnp.float32) + b2_ref[...]


def _z_call(ea, w0, w1, w2, b0, b1, b2):
    zspec = pl.BlockSpec((_EB, D), lambda i: (i, 0))
    wspec = pl.BlockSpec((DE, D), lambda i: (0, 0))
    bspec = pl.BlockSpec((1, D), lambda i: (0, 0))
    return pl.pallas_call(
        _z_body,
        grid=(E // _EB,),
        in_specs=[pl.BlockSpec((_EB, DE), lambda i: (i, 0)),
                  wspec, wspec, wspec, bspec, bspec, bspec],
        out_specs=[zspec, zspec, zspec],
        out_shape=[jax.ShapeDtypeStruct((E, D), jnp.float32)] * 3,
    )(ea, w0, w1, w2, b0, b1, b2)


def _pre_body(h_ref, w1_ref, ws_ref, y_ref, s_ref):
    h = h_ref[...]
    y_ref[...] = jnp.dot(h, w1_ref[...], preferred_element_type=jnp.float32)
    s_ref[...] = jnp.dot(h, ws_ref[...], preferred_element_type=jnp.float32)


def _mid_body(agg_ref, sp_ref, w1_ref, ws_ref, y_ref, s_ref):
    h = jnp.maximum(agg_ref[...] + sp_ref[...], 0.0)
    y_ref[...] = jnp.dot(h, w1_ref[...], preferred_element_type=jnp.float32)
    s_ref[...] = jnp.dot(h, ws_ref[...], preferred_element_type=jnp.float32)


def _h_specs():
    nspec = pl.BlockSpec((_NB, D), lambda i: (i, 0))
    wspec = pl.BlockSpec((D, D), lambda i: (0, 0))
    return nspec, wspec


def _pre_call(h, w1, ws):
    nspec, wspec = _h_specs()
    return pl.pallas_call(
        _pre_body,
        grid=(N // _NB,),
        in_specs=[nspec, wspec, wspec],
        out_specs=[nspec, nspec],
        out_shape=[jax.ShapeDtypeStruct((N, D), jnp.float32)] * 2,
    )(h, w1, ws)


def _mid_call(agg, sp, w1, ws):
    nspec, wspec = _h_specs()
    return pl.pallas_call(
        _mid_body,
        grid=(N // _NB,),
        in_specs=[nspec, nspec, wspec, wspec],
        out_specs=[nspec, nspec],
        out_shape=[jax.ShapeDtypeStruct((N, D), jnp.float32)] * 2,
    )(agg, sp, w1, ws)


_NP = N + 2 * L


def _fin_body(agg_ref, sp_ref, o_ref):
    i = pl.program_id(0)
    h = jnp.maximum(agg_ref[...] + sp_ref[...], 0.0)
    row = i * 128 + lax.broadcasted_iota(jnp.int32, (128, 1), 0)
    o_ref[...] = jnp.where(row < N, h, 0.0)


def _fin_call(agg, sp):
    nspec = pl.BlockSpec((128, D), lambda i: (jnp.minimum(i, 127), 0))
    return pl.pallas_call(
        _fin_body,
        grid=(_NP // 128,),
        in_specs=[nspec, nspec],
        out_specs=pl.BlockSpec((128, D), lambda i: (i, 0)),
        out_shape=jax.ShapeDtypeStruct((_NP, D), jnp.float32),
    )(agg, sp)


def _seq_body(bi_ref, h3_ref, o_ref):
    b = pl.program_id(0)
    bi = bi_ref[...]
    first = jnp.sum((bi < b).astype(jnp.int32))
    cnt = jnp.sum((bi == b).astype(jnp.int32))
    base = pl.multiple_of((first // 8) * 8, 8)
    rem = first - base
    window = h3_ref[pl.ds(base, L + 8), :]
    rows = pltpu.roll(window, (L + 8) - rem, 0)[:L]
    liota = lax.broadcasted_iota(jnp.int32, (L, 1), 0)
    o_ref[0] = jnp.where(liota < cnt, rows, 0.0)


def _seq_call(bi2d, h3p):
    return pl.pallas_call(
        _seq_body,
        grid=(B,),
        in_specs=[pl.BlockSpec((128, 128), lambda b: (0, 0)),
                  pl.BlockSpec((_NP, D), lambda b: (0, 0))],
        out_specs=pl.BlockSpec((1, L, D), lambda b: (b, 0, 0)),
        out_shape=jax.ShapeDtypeStruct((B, L, D), jnp.float32),
    )(bi2d, h3p)


_NW = 32
_OWN = N // 64
_S = 8192
_G = 32
_W = 256
_NBLK = E // _S
_CAP = E + 16 * _NBLK
_RC = 8192

_mesh = plsc.VectorSubcoreMesh(core_axis_name="c", subcore_axis_name="s")


def _prep_body(src_hbm, dst_hbm, rsrc_hbm, reid_hbm, rdl_hbm, cnts_hbm,
               dstb, srcb, cw_s, cw_e, cw_d, cbuf, sem):
    c = lax.axis_index("c")
    s = lax.axis_index("s")
    w = s * 2 + c
    zero16i = jnp.zeros((16,), jnp.int32)
    iota16 = lax.iota(jnp.int32, 16)
    sent16 = jnp.full((16,), _OWN, jnp.int32)

    def zstage(i, _):
        cw_s[pl.ds(i * 16, 16)] = zero16i
        cw_e[pl.ds(i * 16, 16)] = zero16i
        cw_d[pl.ds(i * 16, 16)] = zero16i
        return 0
    lax.fori_loop(0, (_S + _W + 16) // 16, zstage, 0)

    for p in range(2):
        bucket = p * _NW + w
        own_base = bucket * _OWN
        rbase = bucket * _CAP

        def block_body(bi, cur):
            base = bi * _S
            pltpu.sync_copy(dst_hbm.at[pl.ds(base, _S)], dstb)
            pltpu.sync_copy(src_hbm.at[pl.ds(base, _S)], srcb)

            def scan(i, st):
                dv = dstb[pl.ds(i * 16, 16)]
                m = (dv >= own_base) & (dv < own_base + _OWN)
                sv = srcb[pl.ds(i * 16, 16)]
                ev = iota16 + (base + i * 16)
                pos = st + plsc.cumsum(m.astype(jnp.int32)) - 1
                plsc.store_scatter(cw_d, [pos], dv - own_base, mask=m)
                plsc.store_scatter(cw_s, [pos], sv, mask=m)
                plsc.store_scatter(cw_e, [pos], ev, mask=m)
                cnt = plsc.all_reduce_population_count(m)
                return st + cnt[0]

            n = lax.fori_loop(0, _S // 16, scan, 0)
            plsc.store_scatter(cw_d, [n + iota16], sent16)
            plsc.store_scatter(cw_s, [n + iota16], zero16i)
            plsc.store_scatter(cw_e, [n + iota16], zero16i)
            np_ = ((n + 15) // 16) * 16
            nw = (np_ + _W - 1) // _W

            def flush(k, _):
                o = pl.multiple_of(rbase + cur + k * _W, 16)
                pltpu.sync_copy(cw_s.at[pl.ds(k * _W, _W)],
                                rsrc_hbm.at[pl.ds(o, _W)])
                pltpu.sync_copy(cw_e.at[pl.ds(k * _W, _W)],
                                reid_hbm.at[pl.ds(o, _W)])
                pltpu.sync_copy(cw_d.at[pl.ds(k * _W, _W)],
                                rdl_hbm.at[pl.ds(o, _W)])
                return 0

            lax.fori_loop(0, nw, flush, 0)
            return cur + np_

        total = lax.fori_loop(0, _NBLK, block_body, 0)
        cbuf[pl.ds(0, 16)] = jnp.full((16,), total, jnp.int32)
        pltpu.sync_copy(cbuf, cnts_hbm.at[pl.ds(bucket * 16, 16)])


_sc_prep = functools.partial(
    pl.kernel,
    mesh=_mesh,
    compiler_params=pltpu.CompilerParams(needs_layout_passes=False),
    out_type=[
        jax.ShapeDtypeStruct((64 * _CAP,), jnp.int32),
        jax.ShapeDtypeStruct((64 * _CAP,), jnp.int32),
        jax.ShapeDtypeStruct((64 * _CAP,), jnp.int32),
        jax.ShapeDtypeStruct((64 * 16,), jnp.int32),
    ],
    scratch_types=[
        pltpu.VMEM((_S,), jnp.int32),
        pltpu.VMEM((_S,), jnp.int32),
        pltpu.VMEM((_S + _W + 16,), jnp.int32),
        pltpu.VMEM((_S + _W + 16,), jnp.int32),
        pltpu.VMEM((_S + _W + 16,), jnp.int32),
        pltpu.VMEM((16,), jnp.int32),
        pltpu.SemaphoreType.DMA,
    ],
)(_prep_body)


def _layer_body(y_hbm, z_hbm, rsrc_hbm, reid_hbm, rdl_hbm, cnts_hbm, agg_hbm,
                rc_s, rc_e, rc_d, cbuf,
                wrow0, wrow1, yrow0, yrow1, acc,
                semz0, semz1, semy0, semy1):
    c = lax.axis_index("c")
    s = lax.axis_index("s")
    w = s * 2 + c
    zero16f = jnp.zeros((16,), jnp.float32)
    zero16i = jnp.zeros((16,), jnp.int32)
    iota16 = lax.iota(jnp.int32, 16)

    def zrc(i, _):
        rc_s[pl.ds(i * 16, 16)] = zero16i
        rc_e[pl.ds(i * 16, 16)] = zero16i
        rc_d[pl.ds(i * 16, 16)] = zero16i
        return 0
    lax.fori_loop(0, _RC // 16, zrc, 0)

    def zero_acc(i, _):
        for j in range(D // 16):
            acc[i, pl.ds(j * 16, 16)] = zero16f
        return 0

    def issue(goff, wrow, yrow, semz, semy):
        pltpu.async_copy(z_hbm.at[rc_e.at[pl.ds(goff, _G)]], wrow, semz)
        pltpu.async_copy(y_hbm.at[rc_s.at[pl.ds(goff, _G)]], yrow, semy)

    def drain(wrow, yrow, semz, semy):
        pltpu.make_async_copy(z_hbm.at[pl.ds(0, _G)], wrow, semz).wait()
        pltpu.make_async_copy(y_hbm.at[pl.ds(0, _G)], yrow, semy).wait()

    for p in range(2):
        bucket = p * _NW + w
        own_base = bucket * _OWN
        rbase = bucket * _CAP
        pltpu.sync_copy(cnts_hbm.at[pl.ds(bucket * 16, 16)], cbuf)
        cnt = cbuf[pl.ds(0, 16)][0]
        lax.fori_loop(0, _OWN + 1, zero_acc, 0)

        nchunk = (cnt + _RC - 1) // _RC

        def chunk_body(ci, _):
            c0 = ci * _RC
            n_in = jnp.minimum(_RC, cnt - c0)
            f0 = pl.multiple_of(rbase + c0, 16)
            pltpu.sync_copy(rsrc_hbm.at[pl.ds(f0, _RC)], rc_s)
            pltpu.sync_copy(reid_hbm.at[pl.ds(f0, _RC)], rc_e)
            pltpu.sync_copy(rdl_hbm.at[pl.ds(f0, _RC)], rc_d)
            ngrp = (n_in + _G - 1) // _G

            def compute(goff, wrow, yrow):
                nrows = jnp.minimum(_G, n_in - goff)

                def rowfn(i, _):
                    dvi = plsc.load_gather(
                        rc_d, [jnp.full((16,), goff + i, jnp.int32)])
                    for j in range(D // 16):
                        v = wrow[i, pl.ds(j * 16, 16)] + yrow[i, pl.ds(j * 16, 16)]
                        plsc.addupdate_scatter(
                            acc, [dvi, j * 16 + iota16], jnp.maximum(v, 0.0))
                    return 0

                lax.fori_loop(0, nrows, rowfn, 0)

            @pl.when(ngrp > 0)
            def _():
                issue(0, wrow0, yrow0, semz0, semy0)

            def pair(gg, _):
                g0 = 2 * gg
                g1 = g0 + 1

                @pl.when(g1 < ngrp)
                def _():
                    issue(g1 * _G, wrow1, yrow1, semz1, semy1)

                drain(wrow0, yrow0, semz0, semy0)
                compute(g0 * _G, wrow0, yrow0)

                @pl.when(g1 < ngrp)
                def _():
                    @pl.when(g1 + 1 < ngrp)
                    def _():
                        issue((g1 + 1) * _G, wrow0, yrow0, semz0, semy0)

                    drain(wrow1, yrow1, semz1, semy1)
                    compute(g1 * _G, wrow1, yrow1)

                return 0

            lax.fori_loop(0, (ngrp + 1) // 2, pair, 0)
            return 0

        lax.fori_loop(0, nchunk, chunk_body, 0)
        pltpu.sync_copy(acc.at[pl.ds(0, _OWN)],
                        agg_hbm.at[pl.ds(own_base, _OWN)])


_sc_layer = functools.partial(
    pl.kernel,
    mesh=_mesh,
    compiler_params=pltpu.CompilerParams(needs_layout_passes=False),
    out_type=jax.ShapeDtypeStruct((N, D), jnp.float32),
    scratch_types=[
        pltpu.VMEM((_RC,), jnp.int32),
        pltpu.VMEM((_RC,), jnp.int32),
        pltpu.VMEM((_RC,), jnp.int32),
        pltpu.VMEM((16,), jnp.int32),
        pltpu.VMEM((_G, D), jnp.float32),
        pltpu.VMEM((_G, D), jnp.float32),
        pltpu.VMEM((_G, D), jnp.float32),
        pltpu.VMEM((_G, D), jnp.float32),
        pltpu.VMEM((_OWN + 1, D), jnp.float32),
        pltpu.SemaphoreType.DMA,
        pltpu.SemaphoreType.DMA,
        pltpu.SemaphoreType.DMA,
        pltpu.SemaphoreType.DMA,
    ],
)(_layer_body)


def kernel(x, edge_index, edge_attr, pos, batch_indices,
           W1_0, We_0, Ws_0, b_0,
           W1_1, We_1, Ws_1, b_1,
           W1_2, We_2, Ws_2, b_2):
    src = edge_index[0].astype(jnp.int32)
    dst = edge_index[1].astype(jnp.int32)
    bi2d = batch_indices.astype(jnp.int32).reshape(128, 128)

    rsrc, reid, rdl, cnts = _sc_prep(src, dst)
    z0, z1, z2 = _z_call(edge_attr, We_0, We_1, We_2,
                         b_0.reshape(1, D), b_1.reshape(1, D), b_2.reshape(1, D))
    y, sp = _pre_call(x, W1_0, Ws_0)
    agg = _sc_layer(y, z0, rsrc, reid, rdl, cnts)
    y, sp = _mid_call(agg, sp, W1_1, Ws_1)
    agg = _sc_layer(y, z1, rsrc, reid, rdl, cnts)
    y, sp = _mid_call(agg, sp, W1_2, Ws_2)
    agg = _sc_layer(y, z2, rsrc, reid, rdl, cnts)
    h3p = _fin_call(agg, sp)
    return _seq_call(bi2d, h3p)


# speedup vs baseline: 1.3980x; 1.0029x over previous
import functools

import jax
import jax.numpy as jnp
from jax import lax
from jax.experimental import pallas as pl
from jax.experimental.pallas import tpu as pltpu
from jax.experimental.pallas import tpu_sc as plsc

N = 16384
E = 262144
D = 256
DE = 16
B = 256
L = 128


_EB = 2048
_NB = 1024


def _z_body(ea_ref, w0_ref, w1_ref, w2_ref, b0_ref, b1_ref, b2_ref,
            z0_ref, z1_ref, z2_ref):
    ea = ea_ref[...]
    z0_ref[...] = jnp.dot(ea, w0_ref[...], preferred_element_type=jnp.float32) + b0_ref[...]
    z1_ref[...] = jnp.dot(ea, w1_ref[...], preferred_element_type=jnp.float32) + b1_ref[...]
    z2_ref[...] = jnp.dot(ea, w2_ref[...], preferred_element_type=jnp.float32) + b2_ref[...]


def _z_call(ea, w0, w1, w2, b0, b1, b2):
    zspec = pl.BlockSpec((_EB, D), lambda i: (i, 0))
    wspec = pl.BlockSpec((DE, D), lambda i: (0, 0))
    bspec = pl.BlockSpec((1, D), lambda i: (0, 0))
    return pl.pallas_call(
        _z_body,
        grid=(E // _EB,),
        in_specs=[pl.BlockSpec((_EB, DE), lambda i: (i, 0)),
                  wspec, wspec, wspec, bspec, bspec, bspec],
        out_specs=[zspec, zspec, zspec],
        out_shape=[jax.ShapeDtypeStruct((E, D), jnp.float32)] * 3,
    )(ea, w0, w1, w2, b0, b1, b2)


def _pre_body(h_ref, w1_ref, ws_ref, y_ref, s_ref):
    h = h_ref[...]
    y_ref[...] = jnp.dot(h, w1_ref[...], preferred_element_type=jnp.float32)
    s_ref[...] = jnp.dot(h, ws_ref[...], preferred_element_type=jnp.float32)


def _mid_body(agg_ref, sp_ref, w1_ref, ws_ref, y_ref, s_ref):
    h = jnp.maximum(agg_ref[...] + sp_ref[...], 0.0)
    y_ref[...] = jnp.dot(h, w1_ref[...], preferred_element_type=jnp.float32)
    s_ref[...] = jnp.dot(h, ws_ref[...], preferred_element_type=jnp.float32)


def _h_specs():
    nspec = pl.BlockSpec((_NB, D), lambda i: (i, 0))
    wspec = pl.BlockSpec((D, D), lambda i: (0, 0))
    return nspec, wspec


def _pre_call(h, w1, ws):
    nspec, wspec = _h_specs()
    return pl.pallas_call(
        _pre_body,
        grid=(N // _NB,),
        in_specs=[nspec, wspec, wspec],
        out_specs=[nspec, nspec],
        out_shape=[jax.ShapeDtypeStruct((N, D), jnp.float32)] * 2,
    )(h, w1, ws)


def _mid_call(agg, sp, w1, ws):
    nspec, wspec = _h_specs()
    return pl.pallas_call(
        _mid_body,
        grid=(N // _NB,),
        in_specs=[nspec, nspec, wspec, wspec],
        out_specs=[nspec, nspec],
        out_shape=[jax.ShapeDtypeStruct((N, D), jnp.float32)] * 2,
    )(agg, sp, w1, ws)


_NP = N + 2 * L


def _fin_body(agg_ref, sp_ref, o_ref):
    i = pl.program_id(0)
    h = jnp.maximum(agg_ref[...] + sp_ref[...], 0.0)
    row = i * 128 + lax.broadcasted_iota(jnp.int32, (128, 1), 0)
    o_ref[...] = jnp.where(row < N, h, 0.0)


def _fin_call(agg, sp):
    nspec = pl.BlockSpec((128, D), lambda i: (jnp.minimum(i, 127), 0))
    return pl.pallas_call(
        _fin_body,
        grid=(_NP // 128,),
        in_specs=[nspec, nspec],
        out_specs=pl.BlockSpec((128, D), lambda i: (i, 0)),
        out_shape=jax.ShapeDtypeStruct((_NP, D), jnp.float32),
    )(agg, sp)


def _seq_body(bi_ref, h3_ref, o_ref):
    b = pl.program_id(0)
    bi = bi_ref[...]
    first = jnp.sum((bi < b).astype(jnp.int32))
    cnt = jnp.sum((bi == b).astype(jnp.int32))
    base = pl.multiple_of((first // 8) * 8, 8)
    rem = first - base
    window = h3_ref[pl.ds(base, L + 8), :]
    rows = pltpu.roll(window, (L + 8) - rem, 0)[:L]
    liota = lax.broadcasted_iota(jnp.int32, (L, 1), 0)
    o_ref[0] = jnp.where(liota < cnt, rows, 0.0)


def _seq_call(bi2d, h3p):
    return pl.pallas_call(
        _seq_body,
        grid=(B,),
        in_specs=[pl.BlockSpec((128, 128), lambda b: (0, 0)),
                  pl.BlockSpec((_NP, D), lambda b: (0, 0))],
        out_specs=pl.BlockSpec((1, L, D), lambda b: (b, 0, 0)),
        out_shape=jax.ShapeDtypeStruct((B, L, D), jnp.float32),
    )(bi2d, h3p)


_NW = 32
_OWN = N // 64
_S = 8192
_G = 32
_W = 256
_NBLK = E // _S
_CAP = E + 16 * _NBLK
_RC = 8192

_mesh = plsc.VectorSubcoreMesh(core_axis_name="c", subcore_axis_name="s")


def _prep_body(src_hbm, dst_hbm, rsrc_hbm, reid_hbm, rdl_hbm, cnts_hbm,
               dstb, srcb, cw_s, cw_e, cw_d, cbuf, sem):
    c = lax.axis_index("c")
    s = lax.axis_index("s")
    w = s * 2 + c
    zero16i = jnp.zeros((16,), jnp.int32)
    iota16 = lax.iota(jnp.int32, 16)
    sent16 = jnp.full((16,), _OWN, jnp.int32)

    def zstage(i, _):
        cw_s[pl.ds(i * 16, 16)] = zero16i
        cw_e[pl.ds(i * 16, 16)] = zero16i
        cw_d[pl.ds(i * 16, 16)] = zero16i
        return 0
    lax.fori_loop(0, (_S + _W + 16) // 16, zstage, 0)

    for p in range(2):
        bucket = p * _NW + w
        own_base = bucket * _OWN
        rbase = bucket * _CAP

        def block_body(bi, cur):
            base = bi * _S
            pltpu.sync_copy(dst_hbm.at[pl.ds(base, _S)], dstb)
            pltpu.sync_copy(src_hbm.at[pl.ds(base, _S)], srcb)

            def scan(i, st):
                dv = dstb[pl.ds(i * 16, 16)]
                m = (dv >= own_base) & (dv < own_base + _OWN)
                sv = srcb[pl.ds(i * 16, 16)]
                ev = iota16 + (base + i * 16)
                pos = st + plsc.cumsum(m.astype(jnp.int32)) - 1
                plsc.store_scatter(cw_d, [pos], dv - own_base, mask=m)
                plsc.store_scatter(cw_s, [pos], sv, mask=m)
                plsc.store_scatter(cw_e, [pos], ev, mask=m)
                cnt = plsc.all_reduce_population_count(m)
                return st + cnt[0]

            n = lax.fori_loop(0, _S // 16, scan, 0)
            plsc.store_scatter(cw_d, [n + iota16], sent16)
            plsc.store_scatter(cw_s, [n + iota16], zero16i)
            plsc.store_scatter(cw_e, [n + iota16], zero16i)
            np_ = ((n + 15) // 16) * 16
            nw = (np_ + _W - 1) // _W

            def flush(k, _):
                o = pl.multiple_of(rbase + cur + k * _W, 16)
                pltpu.sync_copy(cw_s.at[pl.ds(k * _W, _W)],
                                rsrc_hbm.at[pl.ds(o, _W)])
                pltpu.sync_copy(cw_e.at[pl.ds(k * _W, _W)],
                                reid_hbm.at[pl.ds(o, _W)])
                pltpu.sync_copy(cw_d.at[pl.ds(k * _W, _W)],
                                rdl_hbm.at[pl.ds(o, _W)])
                return 0

            lax.fori_loop(0, nw, flush, 0)
            return cur + np_

        total = lax.fori_loop(0, _NBLK, block_body, 0)
        cbuf[pl.ds(0, 16)] = jnp.full((16,), total, jnp.int32)
        pltpu.sync_copy(cbuf, cnts_hbm.at[pl.ds(bucket * 16, 16)])


_sc_prep = functools.partial(
    pl.kernel,
    mesh=_mesh,
    compiler_params=pltpu.CompilerParams(needs_layout_passes=False),
    out_type=[
        jax.ShapeDtypeStruct((64 * _CAP,), jnp.int32),
        jax.ShapeDtypeStruct((64 * _CAP,), jnp.int32),
        jax.ShapeDtypeStruct((64 * _CAP,), jnp.int32),
        jax.ShapeDtypeStruct((64 * 16,), jnp.int32),
    ],
    scratch_types=[
        pltpu.VMEM((_S,), jnp.int32),
        pltpu.VMEM((_S,), jnp.int32),
        pltpu.VMEM((_S + _W + 16,), jnp.int32),
        pltpu.VMEM((_S + _W + 16,), jnp.int32),
        pltpu.VMEM((_S + _W + 16,), jnp.int32),
        pltpu.VMEM((16,), jnp.int32),
        pltpu.SemaphoreType.DMA,
    ],
)(_prep_body)


def _layer_body(y_hbm, z_hbm, rsrc_hbm, reid_hbm, rdl_hbm, cnts_hbm, agg_hbm,
                rc_s, rc_e, rc_d, cbuf,
                wrow0, wrow1, yrow0, yrow1, acc,
                semz0, semz1, semy0, semy1):
    c = lax.axis_index("c")
    s = lax.axis_index("s")
    w = s * 2 + c
    zero16f = jnp.zeros((16,), jnp.float32)
    zero16i = jnp.zeros((16,), jnp.int32)
    iota16 = lax.iota(jnp.int32, 16)

    def zrc(i, _):
        rc_s[pl.ds(i * 16, 16)] = zero16i
        rc_e[pl.ds(i * 16, 16)] = zero16i
        rc_d[pl.ds(i * 16, 16)] = zero16i
        return 0
    lax.fori_loop(0, _RC // 16, zrc, 0)

    def zero_acc(i, _):
        for j in range(D // 16):
            acc[i, pl.ds(j * 16, 16)] = zero16f
        return 0

    def issue(goff, wrow, yrow, semz, semy):
        pltpu.async_copy(z_hbm.at[rc_e.at[pl.ds(goff, _G)]], wrow, semz)
        pltpu.async_copy(y_hbm.at[rc_s.at[pl.ds(goff, _G)]], yrow, semy)

    def drain(wrow, yrow, semz, semy):
        pltpu.make_async_copy(z_hbm.at[pl.ds(0, _G)], wrow, semz).wait()
        pltpu.make_async_copy(y_hbm.at[pl.ds(0, _G)], yrow, semy).wait()

    for p in range(2):
        bucket = p * _NW + w
        own_base = bucket * _OWN
        rbase = bucket * _CAP
        pltpu.sync_copy(cnts_hbm.at[pl.ds(bucket * 16, 16)], cbuf)
        cnt = cbuf[pl.ds(0, 16)][0]
        lax.fori_loop(0, _OWN + 1, zero_acc, 0)

        nchunk = (cnt + _RC - 1) // _RC

        def chunk_body(ci, _):
            c0 = ci * _RC
            n_in = jnp.minimum(_RC, cnt - c0)
            f0 = pl.multiple_of(rbase + c0, 16)
            pltpu.sync_copy(rsrc_hbm.at[pl.ds(f0, _RC)], rc_s)
            pltpu.sync_copy(reid_hbm.at[pl.ds(f0, _RC)], rc_e)
            pltpu.sync_copy(rdl_hbm.at[pl.ds(f0, _RC)], rc_d)
            ngrp = (n_in + _G - 1) // _G

            def compute(goff, wrow, yrow):
                nrows = jnp.minimum(_G, n_in - goff)

                def rowfn(i4, _):
                    i = i4 * 4
                    dvis = [plsc.load_gather(
                        rc_d, [jnp.full((16,), goff + i + u, jnp.int32)])
                        for u in range(4)]
                    for j in range(D // 16):
                        col = j * 16 + iota16
                        for u in range(4):
                            v = (wrow[i + u, pl.ds(j * 16, 16)]
                                 + yrow[i + u, pl.ds(j * 16, 16)])
                            plsc.addupdate_scatter(
                                acc, [dvis[u], col], jnp.maximum(v, 0.0))
                    return 0

                lax.fori_loop(0, nrows // 4, rowfn, 0)

            @pl.when(ngrp > 0)
            def _():
                issue(0, wrow0, yrow0, semz0, semy0)

            def pair(gg, _):
                g0 = 2 * gg
                g1 = g0 + 1

                @pl.when(g1 < ngrp)
                def _():
                    issue(g1 * _G, wrow1, yrow1, semz1, semy1)

                drain(wrow0, yrow0, semz0, semy0)
                compute(g0 * _G, wrow0, yrow0)

                @pl.when(g1 < ngrp)
                def _():
                    @pl.when(g1 + 1 < ngrp)
                    def _():
                        issue((g1 + 1) * _G, wrow0, yrow0, semz0, semy0)

                    drain(wrow1, yrow1, semz1, semy1)
                    compute(g1 * _G, wrow1, yrow1)

                return 0

            lax.fori_loop(0, (ngrp + 1) // 2, pair, 0)
            return 0

        lax.fori_loop(0, nchunk, chunk_body, 0)
        pltpu.sync_copy(acc.at[pl.ds(0, _OWN)],
                        agg_hbm.at[pl.ds(own_base, _OWN)])


_sc_layer = functools.partial(
    pl.kernel,
    mesh=_mesh,
    compiler_params=pltpu.CompilerParams(needs_layout_passes=False),
    out_type=jax.ShapeDtypeStruct((N, D), jnp.float32),
    scratch_types=[
        pltpu.VMEM((_RC,), jnp.int32),
        pltpu.VMEM((_RC,), jnp.int32),
        pltpu.VMEM((_RC,), jnp.int32),
        pltpu.VMEM((16,), jnp.int32),
        pltpu.VMEM((_G, D), jnp.float32),
        pltpu.VMEM((_G, D), jnp.float32),
        pltpu.VMEM((_G, D), jnp.float32),
        pltpu.VMEM((_G, D), jnp.float32),
        pltpu.VMEM((_OWN + 1, D), jnp.float32),
        pltpu.SemaphoreType.DMA,
        pltpu.SemaphoreType.DMA,
        pltpu.SemaphoreType.DMA,
        pltpu.SemaphoreType.DMA,
    ],
)(_layer_body)


def kernel(x, edge_index, edge_attr, pos, batch_indices,
           W1_0, We_0, Ws_0, b_0,
           W1_1, We_1, Ws_1, b_1,
           W1_2, We_2, Ws_2, b_2):
    src = edge_index[0].astype(jnp.int32)
    dst = edge_index[1].astype(jnp.int32)
    bi2d = batch_indices.astype(jnp.int32).reshape(128, 128)

    rsrc, reid, rdl, cnts = _sc_prep(src, dst)
    z0, z1, z2 = _z_call(edge_attr, We_0, We_1, We_2,
                         b_0.reshape(1, D), b_1.reshape(1, D), b_2.reshape(1, D))
    y, sp = _pre_call(x, W1_0, Ws_0)
    agg = _sc_layer(y, z0, rsrc, reid, rdl, cnts)
    y, sp = _mid_call(agg, sp, W1_1, Ws_1)
    agg = _sc_layer(y, z1, rsrc, reid, rdl, cnts)
    y, sp = _mid_call(agg, sp, W1_2, Ws_2)
    agg = _sc_layer(y, z2, rsrc, reid, rdl, cnts)
    h3p = _fin_call(agg, sp)
    return _seq_call(bi2d, h3p)


# speedup vs baseline: 1.4347x; 1.0263x over previous
import functools

import jax
import jax.numpy as jnp
from jax import lax
from jax.experimental import pallas as pl
from jax.experimental.pallas import tpu as pltpu
from jax.experimental.pallas import tpu_sc as plsc

N = 16384
E = 262144
D = 256
DE = 16
B = 256
L = 128


_EB = 2048
_NB = 1024


def _z_body(ea_ref, w0_ref, w1_ref, w2_ref, b0_ref, b1_ref, b2_ref,
            z0_ref, z1_ref, z2_ref):
    ea = ea_ref[...]
    z0_ref[...] = jnp.dot(ea, w0_ref[...], preferred_element_type=jnp.float32) + b0_ref[...]
    z1_ref[...] = jnp.dot(ea, w1_ref[...], preferred_element_type=jnp.float32) + b1_ref[...]
    z2_ref[...] = jnp.dot(ea, w2_ref[...], preferred_element_type=jnp.float32) + b2_ref[...]


def _z_call(ea, w0, w1, w2, b0, b1, b2):
    zspec = pl.BlockSpec((_EB, D), lambda i: (i, 0))
    wspec = pl.BlockSpec((DE, D), lambda i: (0, 0))
    bspec = pl.BlockSpec((1, D), lambda i: (0, 0))
    return pl.pallas_call(
        _z_body,
        grid=(E // _EB,),
        in_specs=[pl.BlockSpec((_EB, DE), lambda i: (i, 0)),
                  wspec, wspec, wspec, bspec, bspec, bspec],
        out_specs=[zspec, zspec, zspec],
        out_shape=[jax.ShapeDtypeStruct((E, D), jnp.float32)] * 3,
    )(ea, w0, w1, w2, b0, b1, b2)


def _pre_body(h_ref, w1_ref, ws_ref, y_ref, s_ref):
    h = h_ref[...]
    y_ref[...] = jnp.dot(h, w1_ref[...], preferred_element_type=jnp.float32)
    s_ref[...] = jnp.dot(h, ws_ref[...], preferred_element_type=jnp.float32)


def _mid_body(agg_ref, sp_ref, w1_ref, ws_ref, y_ref, s_ref):
    h = jnp.maximum(agg_ref[...] + sp_ref[...], 0.0)
    y_ref[...] = jnp.dot(h, w1_ref[...], preferred_element_type=jnp.float32)
    s_ref[...] = jnp.dot(h, ws_ref[...], preferred_element_type=jnp.float32)


def _h_specs():
    nspec = pl.BlockSpec((_NB, D), lambda i: (i, 0))
    wspec = pl.BlockSpec((D, D), lambda i: (0, 0))
    return nspec, wspec


def _pre_call(h, w1, ws):
    nspec, wspec = _h_specs()
    return pl.pallas_call(
        _pre_body,
        grid=(N // _NB,),
        in_specs=[nspec, wspec, wspec],
        out_specs=[nspec, nspec],
        out_shape=[jax.ShapeDtypeStruct((N, D), jnp.float32)] * 2,
    )(h, w1, ws)


def _mid_call(agg, sp, w1, ws):
    nspec, wspec = _h_specs()
    return pl.pallas_call(
        _mid_body,
        grid=(N // _NB,),
        in_specs=[nspec, nspec, wspec, wspec],
        out_specs=[nspec, nspec],
        out_shape=[jax.ShapeDtypeStruct((N, D), jnp.float32)] * 2,
    )(agg, sp, w1, ws)


_NP = N + 2 * L


def _fin_body(agg_ref, sp_ref, o_ref):
    i = pl.program_id(0)
    h = jnp.maximum(agg_ref[...] + sp_ref[...], 0.0)
    row = i * 128 + lax.broadcasted_iota(jnp.int32, (128, 1), 0)
    o_ref[...] = jnp.where(row < N, h, 0.0)


def _fin_call(agg, sp):
    nspec = pl.BlockSpec((128, D), lambda i: (jnp.minimum(i, 127), 0))
    return pl.pallas_call(
        _fin_body,
        grid=(_NP // 128,),
        in_specs=[nspec, nspec],
        out_specs=pl.BlockSpec((128, D), lambda i: (i, 0)),
        out_shape=jax.ShapeDtypeStruct((_NP, D), jnp.float32),
    )(agg, sp)


def _seq_body(bi_ref, h3_ref, o_ref):
    b = pl.program_id(0)
    bi = bi_ref[...]
    first = jnp.sum((bi < b).astype(jnp.int32))
    cnt = jnp.sum((bi == b).astype(jnp.int32))
    base = pl.multiple_of((first // 8) * 8, 8)
    rem = first - base
    window = h3_ref[pl.ds(base, L + 8), :]
    rows = pltpu.roll(window, (L + 8) - rem, 0)[:L]
    liota = lax.broadcasted_iota(jnp.int32, (L, 1), 0)
    o_ref[0] = jnp.where(liota < cnt, rows, 0.0)


def _seq_call(bi2d, h3p):
    return pl.pallas_call(
        _seq_body,
        grid=(B,),
        in_specs=[pl.BlockSpec((128, 128), lambda b: (0, 0)),
                  pl.BlockSpec((_NP, D), lambda b: (0, 0))],
        out_specs=pl.BlockSpec((1, L, D), lambda b: (b, 0, 0)),
        out_shape=jax.ShapeDtypeStruct((B, L, D), jnp.float32),
    )(bi2d, h3p)


_NW = 32
_OWN = N // 64
_S = 8192
_G = 32
_W = 256
_NBLK = E // _S
_CAP = E + 16 * _NBLK
_RC = 8192

_mesh = plsc.VectorSubcoreMesh(core_axis_name="c", subcore_axis_name="s")


def _prep_body(src_hbm, dst_hbm, rsrc_hbm, reid_hbm, rdl_hbm, cnts_hbm,
               dstb, srcb, cw_s, cw_e, cw_d, cbuf, sem):
    c = lax.axis_index("c")
    s = lax.axis_index("s")
    w = s * 2 + c
    zero16i = jnp.zeros((16,), jnp.int32)
    iota16 = lax.iota(jnp.int32, 16)
    sent16 = jnp.full((16,), _OWN, jnp.int32)

    def zstage(i, _):
        cw_s[pl.ds(i * 16, 16)] = zero16i
        cw_e[pl.ds(i * 16, 16)] = zero16i
        cw_d[pl.ds(i * 16, 16)] = zero16i
        return 0
    lax.fori_loop(0, (_S + _W + 16) // 16, zstage, 0)

    for p in range(2):
        bucket = p * _NW + w
        own_base = bucket * _OWN
        rbase = bucket * _CAP

        def block_body(bi, cur):
            base = bi * _S
            pltpu.sync_copy(dst_hbm.at[pl.ds(base, _S)], dstb)
            pltpu.sync_copy(src_hbm.at[pl.ds(base, _S)], srcb)

            def scan(i, st):
                dv = dstb[pl.ds(i * 16, 16)]
                m = (dv >= own_base) & (dv < own_base + _OWN)
                sv = srcb[pl.ds(i * 16, 16)]
                ev = iota16 + (base + i * 16)
                pos = st + plsc.cumsum(m.astype(jnp.int32)) - 1
                plsc.store_scatter(cw_d, [pos], dv - own_base, mask=m)
                plsc.store_scatter(cw_s, [pos], sv, mask=m)
                plsc.store_scatter(cw_e, [pos], ev, mask=m)
                cnt = plsc.all_reduce_population_count(m)
                return st + cnt[0]

            n = lax.fori_loop(0, _S // 16, scan, 0)
            plsc.store_scatter(cw_d, [n + iota16], sent16)
            plsc.store_scatter(cw_s, [n + iota16], zero16i)
            plsc.store_scatter(cw_e, [n + iota16], zero16i)
            np_ = ((n + 15) // 16) * 16
            nw = (np_ + _W - 1) // _W

            def flush(k, _):
                o = pl.multiple_of(rbase + cur + k * _W, 16)
                pltpu.sync_copy(cw_s.at[pl.ds(k * _W, _W)],
                                rsrc_hbm.at[pl.ds(o, _W)])
                pltpu.sync_copy(cw_e.at[pl.ds(k * _W, _W)],
                                reid_hbm.at[pl.ds(o, _W)])
                pltpu.sync_copy(cw_d.at[pl.ds(k * _W, _W)],
                                rdl_hbm.at[pl.ds(o, _W)])
                return 0

            lax.fori_loop(0, nw, flush, 0)
            return cur + np_

        total = lax.fori_loop(0, _NBLK, block_body, 0)
        cbuf[pl.ds(0, 16)] = jnp.full((16,), total, jnp.int32)
        pltpu.sync_copy(cbuf, cnts_hbm.at[pl.ds(bucket * 16, 16)])


_sc_prep = functools.partial(
    pl.kernel,
    mesh=_mesh,
    compiler_params=pltpu.CompilerParams(needs_layout_passes=False),
    out_type=[
        jax.ShapeDtypeStruct((64 * _CAP,), jnp.int32),
        jax.ShapeDtypeStruct((64 * _CAP,), jnp.int32),
        jax.ShapeDtypeStruct((64 * _CAP,), jnp.int32),
        jax.ShapeDtypeStruct((64 * 16,), jnp.int32),
    ],
    scratch_types=[
        pltpu.VMEM((_S,), jnp.int32),
        pltpu.VMEM((_S,), jnp.int32),
        pltpu.VMEM((_S + _W + 16,), jnp.int32),
        pltpu.VMEM((_S + _W + 16,), jnp.int32),
        pltpu.VMEM((_S + _W + 16,), jnp.int32),
        pltpu.VMEM((16,), jnp.int32),
        pltpu.SemaphoreType.DMA,
    ],
)(_prep_body)


def _layer_body(y_hbm, z_hbm, rsrc_hbm, reid_hbm, rdl_hbm, cnts_hbm, agg_hbm,
                rc_s, rc_e, rc_d, cbuf, w0, w1, w2, w3, acc,
                sz0, sz1, sz2, sz3, sy0, sy1, sy2, sy3):
    c = lax.axis_index("c")
    s = lax.axis_index("s")
    w = s * 2 + c
    zero16f = jnp.zeros((16,), jnp.float32)
    zero16i = jnp.zeros((16,), jnp.int32)
    iota16 = lax.iota(jnp.int32, 16)
    wbuf = (w0, w1, w2, w3)
    semz = (sz0, sz1, sz2, sz3)
    semy = (sy0, sy1, sy2, sy3)

    def zrc(i, _):
        rc_s[pl.ds(i * 16, 16)] = zero16i
        rc_e[pl.ds(i * 16, 16)] = zero16i
        rc_d[pl.ds(i * 16, 16)] = zero16i
        return 0
    lax.fori_loop(0, _RC // 16, zrc, 0)

    def zero_acc(i, _):
        for j in range(D // 16):
            acc[i, pl.ds(j * 16, 16)] = zero16f
        return 0

    for p in range(2):
        bucket = p * _NW + w
        own_base = bucket * _OWN
        rbase = bucket * _CAP
        pltpu.sync_copy(cnts_hbm.at[pl.ds(bucket * 16, 16)], cbuf)
        cnt = cbuf[pl.ds(0, 16)][0]
        lax.fori_loop(0, _OWN + 1, zero_acc, 0)

        nchunk = (cnt + _RC - 1) // _RC

        def chunk_body(ci, _):
            c0 = ci * _RC
            n_in = jnp.minimum(_RC, cnt - c0)
            f0 = pl.multiple_of(rbase + c0, 16)
            pltpu.sync_copy(rsrc_hbm.at[pl.ds(f0, _RC)], rc_s)
            pltpu.sync_copy(reid_hbm.at[pl.ds(f0, _RC)], rc_e)
            pltpu.sync_copy(rdl_hbm.at[pl.ds(f0, _RC)], rc_d)
            ngrp = (n_in + _G - 1) // _G

            def zissue(u, g):
                pltpu.async_copy(
                    z_hbm.at[rc_e.at[pl.ds(g * _G, _G)]], wbuf[u], semz[u])

            def yissue(u, g):
                pltpu.async_copy(
                    y_hbm.at[rc_s.at[pl.ds(g * _G, _G)]], wbuf[u], semy[u],
                    add=True)

            def zdrain(u):
                pltpu.make_async_copy(
                    z_hbm.at[pl.ds(0, _G)], wbuf[u], semz[u]).wait()

            def ydrain(u):
                pltpu.make_async_copy(
                    y_hbm.at[pl.ds(0, _G)], wbuf[u], semy[u]).wait()

            def compute(goff, wrow):
                nrows = jnp.minimum(_G, n_in - goff)

                def rowfn(i4, _):
                    i = i4 * 4
                    dvis = [plsc.load_gather(
                        rc_d, [jnp.full((16,), goff + i + u, jnp.int32)])
                        for u in range(4)]
                    for j in range(D // 16):
                        col = j * 16 + iota16
                        for u in range(4):
                            v = wrow[i + u, pl.ds(j * 16, 16)]
                            plsc.addupdate_scatter(
                                acc, [dvis[u], col], jnp.maximum(v, 0.0))
                    return 0

                lax.fori_loop(0, nrows // 4, rowfn, 0)

            for u in range(4):
                @pl.when(u < ngrp)
                def _(u=u):
                    zissue(u, u)

            def quad(q, _):
                g0 = q * 4
                for u in range(4):
                    @pl.when(g0 + u < ngrp)
                    def _(u=u):
                        zdrain(u)
                        yissue(u, g0 + u)
                for u in range(4):
                    @pl.when(g0 + u < ngrp)
                    def _(u=u):
                        ydrain(u)
                        compute((g0 + u) * _G, wbuf[u])

                        @pl.when(g0 + u + 4 < ngrp)
                        def _():
                            zissue(u, g0 + u + 4)
                return 0

            lax.fori_loop(0, (ngrp + 3) // 4, quad, 0)
            return 0

        lax.fori_loop(0, nchunk, chunk_body, 0)
        pltpu.sync_copy(acc.at[pl.ds(0, _OWN)],
                        agg_hbm.at[pl.ds(own_base, _OWN)])


_sc_layer = functools.partial(
    pl.kernel,
    mesh=_mesh,
    compiler_params=pltpu.CompilerParams(needs_layout_passes=False),
    out_type=jax.ShapeDtypeStruct((N, D), jnp.float32),
    scratch_types=[
        pltpu.VMEM((_RC,), jnp.int32),
        pltpu.VMEM((_RC,), jnp.int32),
        pltpu.VMEM((_RC,), jnp.int32),
        pltpu.VMEM((16,), jnp.int32),
        pltpu.VMEM((_G, D), jnp.float32),
        pltpu.VMEM((_G, D), jnp.float32),
        pltpu.VMEM((_G, D), jnp.float32),
        pltpu.VMEM((_G, D), jnp.float32),
        pltpu.VMEM((_OWN + 1, D), jnp.float32),
        pltpu.SemaphoreType.DMA,
        pltpu.SemaphoreType.DMA,
        pltpu.SemaphoreType.DMA,
        pltpu.SemaphoreType.DMA,
        pltpu.SemaphoreType.DMA,
        pltpu.SemaphoreType.DMA,
        pltpu.SemaphoreType.DMA,
        pltpu.SemaphoreType.DMA,
    ],
)(_layer_body)


def kernel(x, edge_index, edge_attr, pos, batch_indices,
           W1_0, We_0, Ws_0, b_0,
           W1_1, We_1, Ws_1, b_1,
           W1_2, We_2, Ws_2, b_2):
    src = edge_index[0].astype(jnp.int32)
    dst = edge_index[1].astype(jnp.int32)
    bi2d = batch_indices.astype(jnp.int32).reshape(128, 128)

    rsrc, reid, rdl, cnts = _sc_prep(src, dst)
    z0, z1, z2 = _z_call(edge_attr, We_0, We_1, We_2,
                         b_0.reshape(1, D), b_1.reshape(1, D), b_2.reshape(1, D))
    y, sp = _pre_call(x, W1_0, Ws_0)
    agg = _sc_layer(y, z0, rsrc, reid, rdl, cnts)
    y, sp = _mid_call(agg, sp, W1_1, Ws_1)
    agg = _sc_layer(y, z1, rsrc, reid, rdl, cnts)
    y, sp = _mid_call(agg, sp, W1_2, Ws_2)
    agg = _sc_layer(y, z2, rsrc, reid, rdl, cnts)
    h3p = _fin_call(agg, sp)
    return _seq_call(bi2d, h3p)


# speedup vs baseline: 1.6084x; 1.1211x over previous
import functools

import jax
import jax.numpy as jnp
from jax import lax
from jax.experimental import pallas as pl
from jax.experimental.pallas import tpu as pltpu
from jax.experimental.pallas import tpu_sc as plsc

N = 16384
E = 262144
D = 256
DE = 16
B = 256
L = 128


_EB = 2048
_NB = 1024


def _pack_bf16(x):
    xb = x.astype(jnp.bfloat16).astype(jnp.float32)
    lo = jax.lax.bitcast_convert_type(xb[:, :D // 2], jnp.int32)
    hi = jax.lax.bitcast_convert_type(xb[:, D // 2:], jnp.int32)
    himask = jnp.int32(-65536)
    return jax.lax.shift_right_logical(lo, 16) | (hi & himask)


def _z_body(ea_ref, w0_ref, w1_ref, w2_ref, b0_ref, b1_ref, b2_ref,
            z0_ref, z1_ref, z2_ref):
    ea = ea_ref[...]
    z0_ref[...] = _pack_bf16(
        jnp.dot(ea, w0_ref[...], preferred_element_type=jnp.float32) + b0_ref[...])
    z1_ref[...] = _pack_bf16(
        jnp.dot(ea, w1_ref[...], preferred_element_type=jnp.float32) + b1_ref[...])
    z2_ref[...] = _pack_bf16(
        jnp.dot(ea, w2_ref[...], preferred_element_type=jnp.float32) + b2_ref[...])


def _z_call(ea, w0, w1, w2, b0, b1, b2):
    zspec = pl.BlockSpec((_EB, D // 2), lambda i: (i, 0))
    wspec = pl.BlockSpec((DE, D), lambda i: (0, 0))
    bspec = pl.BlockSpec((1, D), lambda i: (0, 0))
    return pl.pallas_call(
        _z_body,
        grid=(E // _EB,),
        in_specs=[pl.BlockSpec((_EB, DE), lambda i: (i, 0)),
                  wspec, wspec, wspec, bspec, bspec, bspec],
        out_specs=[zspec, zspec, zspec],
        out_shape=[jax.ShapeDtypeStruct((E, D // 2), jnp.int32)] * 3,
    )(ea, w0, w1, w2, b0, b1, b2)


def _pre_body(h_ref, w1_ref, ws_ref, y_ref, s_ref):
    h = h_ref[...]
    y_ref[...] = _pack_bf16(
        jnp.dot(h, w1_ref[...], preferred_element_type=jnp.float32))
    s_ref[...] = jnp.dot(h, ws_ref[...], preferred_element_type=jnp.float32)


def _mid_body(agg_ref, sp_ref, w1_ref, ws_ref, y_ref, s_ref):
    h = jnp.maximum(agg_ref[...] + sp_ref[...], 0.0)
    y_ref[...] = _pack_bf16(
        jnp.dot(h, w1_ref[...], preferred_element_type=jnp.float32))
    s_ref[...] = jnp.dot(h, ws_ref[...], preferred_element_type=jnp.float32)


def _h_specs():
    nspec = pl.BlockSpec((_NB, D), lambda i: (i, 0))
    yspec = pl.BlockSpec((_NB, D // 2), lambda i: (i, 0))
    wspec = pl.BlockSpec((D, D), lambda i: (0, 0))
    return nspec, yspec, wspec


def _pre_call(h, w1, ws):
    nspec, yspec, wspec = _h_specs()
    return pl.pallas_call(
        _pre_body,
        grid=(N // _NB,),
        in_specs=[nspec, wspec, wspec],
        out_specs=[yspec, nspec],
        out_shape=[jax.ShapeDtypeStruct((N, D // 2), jnp.int32),
                   jax.ShapeDtypeStruct((N, D), jnp.float32)],
    )(h, w1, ws)


def _mid_call(agg, sp, w1, ws):
    nspec, yspec, wspec = _h_specs()
    return pl.pallas_call(
        _mid_body,
        grid=(N // _NB,),
        in_specs=[nspec, nspec, wspec, wspec],
        out_specs=[yspec, nspec],
        out_shape=[jax.ShapeDtypeStruct((N, D // 2), jnp.int32),
                   jax.ShapeDtypeStruct((N, D), jnp.float32)],
    )(agg, sp, w1, ws)


_NP = N + 2 * L


def _fin_body(agg_ref, sp_ref, o_ref):
    i = pl.program_id(0)
    h = jnp.maximum(agg_ref[...] + sp_ref[...], 0.0)
    row = i * 128 + lax.broadcasted_iota(jnp.int32, (128, 1), 0)
    o_ref[...] = jnp.where(row < N, h, 0.0)


def _fin_call(agg, sp):
    nspec = pl.BlockSpec((128, D), lambda i: (jnp.minimum(i, 127), 0))
    return pl.pallas_call(
        _fin_body,
        grid=(_NP // 128,),
        in_specs=[nspec, nspec],
        out_specs=pl.BlockSpec((128, D), lambda i: (i, 0)),
        out_shape=jax.ShapeDtypeStruct((_NP, D), jnp.float32),
    )(agg, sp)


def _seq_body(bi_ref, h3_ref, o_ref):
    b = pl.program_id(0)
    bi = bi_ref[...]
    first = jnp.sum((bi < b).astype(jnp.int32))
    cnt = jnp.sum((bi == b).astype(jnp.int32))
    base = pl.multiple_of((first // 8) * 8, 8)
    rem = first - base
    window = h3_ref[pl.ds(base, L + 8), :]
    rows = pltpu.roll(window, (L + 8) - rem, 0)[:L]
    liota = lax.broadcasted_iota(jnp.int32, (L, 1), 0)
    o_ref[0] = jnp.where(liota < cnt, rows, 0.0)


def _seq_call(bi2d, h3p):
    return pl.pallas_call(
        _seq_body,
        grid=(B,),
        in_specs=[pl.BlockSpec((128, 128), lambda b: (0, 0)),
                  pl.BlockSpec((_NP, D), lambda b: (0, 0))],
        out_specs=pl.BlockSpec((1, L, D), lambda b: (b, 0, 0)),
        out_shape=jax.ShapeDtypeStruct((B, L, D), jnp.float32),
    )(bi2d, h3p)


_NW = 32
_OWN = N // 64
_S = 8192
_G = 32
_W = 256
_NBLK = E // _S
_CAP = E + 16 * _NBLK
_RC = 8192

_mesh = plsc.VectorSubcoreMesh(core_axis_name="c", subcore_axis_name="s")


def _prep_body(src_hbm, dst_hbm, rsrc_hbm, reid_hbm, rdl_hbm, cnts_hbm,
               dstb, srcb, cw_s, cw_e, cw_d, cbuf, sem):
    c = lax.axis_index("c")
    s = lax.axis_index("s")
    w = s * 2 + c
    zero16i = jnp.zeros((16,), jnp.int32)
    iota16 = lax.iota(jnp.int32, 16)
    sent16 = jnp.full((16,), _OWN, jnp.int32)

    def zstage(i, _):
        cw_s[pl.ds(i * 16, 16)] = zero16i
        cw_e[pl.ds(i * 16, 16)] = zero16i
        cw_d[pl.ds(i * 16, 16)] = zero16i
        return 0
    lax.fori_loop(0, (_S + _W + 16) // 16, zstage, 0)

    for p in range(2):
        bucket = p * _NW + w
        own_base = bucket * _OWN
        rbase = bucket * _CAP

        def block_body(bi, cur):
            base = bi * _S
            pltpu.sync_copy(dst_hbm.at[pl.ds(base, _S)], dstb)
            pltpu.sync_copy(src_hbm.at[pl.ds(base, _S)], srcb)

            def scan(i, st):
                dv = dstb[pl.ds(i * 16, 16)]
                m = (dv >= own_base) & (dv < own_base + _OWN)
                sv = srcb[pl.ds(i * 16, 16)]
                ev = iota16 + (base + i * 16)
                pos = st + plsc.cumsum(m.astype(jnp.int32)) - 1
                plsc.store_scatter(cw_d, [pos], dv - own_base, mask=m)
                plsc.store_scatter(cw_s, [pos], sv, mask=m)
                plsc.store_scatter(cw_e, [pos], ev, mask=m)
                cnt = plsc.all_reduce_population_count(m)
                return st + cnt[0]

            n = lax.fori_loop(0, _S // 16, scan, 0)
            plsc.store_scatter(cw_d, [n + iota16], sent16)
            plsc.store_scatter(cw_s, [n + iota16], zero16i)
            plsc.store_scatter(cw_e, [n + iota16], zero16i)
            np_ = ((n + 15) // 16) * 16
            nw = (np_ + _W - 1) // _W

            def flush(k, _):
                o = pl.multiple_of(rbase + cur + k * _W, 16)
                pltpu.sync_copy(cw_s.at[pl.ds(k * _W, _W)],
                                rsrc_hbm.at[pl.ds(o, _W)])
                pltpu.sync_copy(cw_e.at[pl.ds(k * _W, _W)],
                                reid_hbm.at[pl.ds(o, _W)])
                pltpu.sync_copy(cw_d.at[pl.ds(k * _W, _W)],
                                rdl_hbm.at[pl.ds(o, _W)])
                return 0

            lax.fori_loop(0, nw, flush, 0)
            return cur + np_

        total = lax.fori_loop(0, _NBLK, block_body, 0)
        cbuf[pl.ds(0, 16)] = jnp.full((16,), total, jnp.int32)
        pltpu.sync_copy(cbuf, cnts_hbm.at[pl.ds(bucket * 16, 16)])


_sc_prep = functools.partial(
    pl.kernel,
    mesh=_mesh,
    compiler_params=pltpu.CompilerParams(needs_layout_passes=False),
    out_type=[
        jax.ShapeDtypeStruct((64 * _CAP,), jnp.int32),
        jax.ShapeDtypeStruct((64 * _CAP,), jnp.int32),
        jax.ShapeDtypeStruct((64 * _CAP,), jnp.int32),
        jax.ShapeDtypeStruct((64 * 16,), jnp.int32),
    ],
    scratch_types=[
        pltpu.VMEM((_S,), jnp.int32),
        pltpu.VMEM((_S,), jnp.int32),
        pltpu.VMEM((_S + _W + 16,), jnp.int32),
        pltpu.VMEM((_S + _W + 16,), jnp.int32),
        pltpu.VMEM((_S + _W + 16,), jnp.int32),
        pltpu.VMEM((16,), jnp.int32),
        pltpu.SemaphoreType.DMA,
    ],
)(_prep_body)


def _layer_body(y_hbm, z_hbm, rsrc_hbm, reid_hbm, rdl_hbm, cnts_hbm, agg_hbm,
                rc_s, rc_e, rc_d, cbuf,
                w0, w1, w2, w3, v0, v1, v2, v3, acc,
                sz0, sz1, sz2, sz3, sy0, sy1, sy2, sy3):
    c = lax.axis_index("c")
    s = lax.axis_index("s")
    w = s * 2 + c
    zero16f = jnp.zeros((16,), jnp.float32)
    zero16i = jnp.zeros((16,), jnp.int32)
    iota16 = lax.iota(jnp.int32, 16)
    wbuf = (w0, w1, w2, w3)
    ybuf = (v0, v1, v2, v3)
    semz = (sz0, sz1, sz2, sz3)
    semy = (sy0, sy1, sy2, sy3)
    himask = jnp.full((16,), -65536, jnp.int32)

    def zrc(i, _):
        rc_s[pl.ds(i * 16, 16)] = zero16i
        rc_e[pl.ds(i * 16, 16)] = zero16i
        rc_d[pl.ds(i * 16, 16)] = zero16i
        return 0
    lax.fori_loop(0, _RC // 16, zrc, 0)

    def zero_acc(i, _):
        for j in range(D // 16):
            acc[i, pl.ds(j * 16, 16)] = zero16f
        return 0

    for p in range(2):
        bucket = p * _NW + w
        own_base = bucket * _OWN
        rbase = bucket * _CAP
        pltpu.sync_copy(cnts_hbm.at[pl.ds(bucket * 16, 16)], cbuf)
        cnt = cbuf[pl.ds(0, 16)][0]
        lax.fori_loop(0, _OWN + 1, zero_acc, 0)

        nchunk = (cnt + _RC - 1) // _RC

        def chunk_body(ci, _):
            c0 = ci * _RC
            n_in = jnp.minimum(_RC, cnt - c0)
            f0 = pl.multiple_of(rbase + c0, 16)
            pltpu.sync_copy(rsrc_hbm.at[pl.ds(f0, _RC)], rc_s)
            pltpu.sync_copy(reid_hbm.at[pl.ds(f0, _RC)], rc_e)
            pltpu.sync_copy(rdl_hbm.at[pl.ds(f0, _RC)], rc_d)
            ngrp = (n_in + _G - 1) // _G

            def issue(u, g):
                pltpu.async_copy(
                    z_hbm.at[rc_e.at[pl.ds(g * _G, _G)]], wbuf[u], semz[u])
                pltpu.async_copy(
                    y_hbm.at[rc_s.at[pl.ds(g * _G, _G)]], ybuf[u], semy[u])

            def drain(u):
                pltpu.make_async_copy(
                    z_hbm.at[pl.ds(0, _G)], wbuf[u], semz[u]).wait()
                pltpu.make_async_copy(
                    y_hbm.at[pl.ds(0, _G)], ybuf[u], semy[u]).wait()

            def compute(goff, wrow, yrow):
                nrows = jnp.minimum(_G, n_in - goff)

                def rowfn(i2, _):
                    i = i2 * 2
                    dvis = [plsc.load_gather(
                        rc_d, [jnp.full((16,), goff + i + u, jnp.int32)])
                        for u in range(2)]
                    for j in range(D // 32):
                        cola = j * 16 + iota16
                        colb = (D // 2) + j * 16 + iota16
                        for u in range(2):
                            zw = wrow[i + u, pl.ds(j * 16, 16)]
                            yw = yrow[i + u, pl.ds(j * 16, 16)]
                            za = plsc.bitcast(zw << 16, jnp.float32)
                            ya = plsc.bitcast(yw << 16, jnp.float32)
                            zb = plsc.bitcast(zw & himask, jnp.float32)
                            yb = plsc.bitcast(yw & himask, jnp.float32)
                            plsc.addupdate_scatter(
                                acc, [dvis[u], cola],
                                jnp.maximum(za + ya, 0.0))
                            plsc.addupdate_scatter(
                                acc, [dvis[u], colb],
                                jnp.maximum(zb + yb, 0.0))
                    return 0

                lax.fori_loop(0, nrows // 2, rowfn, 0)

            for u in range(4):
                @pl.when(u < ngrp)
                def _(u=u):
                    issue(u, u)

            def quad(q, _):
                g0 = q * 4
                for u in range(4):
                    @pl.when(g0 + u < ngrp)
                    def _(u=u):
                        drain(u)
                        compute((g0 + u) * _G, wbuf[u], ybuf[u])

                        @pl.when(g0 + u + 4 < ngrp)
                        def _():
                            issue(u, g0 + u + 4)
                return 0

            lax.fori_loop(0, (ngrp + 3) // 4, quad, 0)
            return 0

        lax.fori_loop(0, nchunk, chunk_body, 0)
        pltpu.sync_copy(acc.at[pl.ds(0, _OWN)],
                        agg_hbm.at[pl.ds(own_base, _OWN)])


_sc_layer = functools.partial(
    pl.kernel,
    mesh=_mesh,
    compiler_params=pltpu.CompilerParams(needs_layout_passes=False),
    out_type=jax.ShapeDtypeStruct((N, D), jnp.float32),
    scratch_types=[
        pltpu.VMEM((_RC,), jnp.int32),
        pltpu.VMEM((_RC,), jnp.int32),
        pltpu.VMEM((_RC,), jnp.int32),
        pltpu.VMEM((16,), jnp.int32),
        pltpu.VMEM((_G, D // 2), jnp.int32),
        pltpu.VMEM((_G, D // 2), jnp.int32),
        pltpu.VMEM((_G, D // 2), jnp.int32),
        pltpu.VMEM((_G, D // 2), jnp.int32),
        pltpu.VMEM((_G, D // 2), jnp.int32),
        pltpu.VMEM((_G, D // 2), jnp.int32),
        pltpu.VMEM((_G, D // 2), jnp.int32),
        pltpu.VMEM((_G, D // 2), jnp.int32),
        pltpu.VMEM((_OWN + 1, D), jnp.float32),
        pltpu.SemaphoreType.DMA,
        pltpu.SemaphoreType.DMA,
        pltpu.SemaphoreType.DMA,
        pltpu.SemaphoreType.DMA,
        pltpu.SemaphoreType.DMA,
        pltpu.SemaphoreType.DMA,
        pltpu.SemaphoreType.DMA,
        pltpu.SemaphoreType.DMA,
    ],
)(_layer_body)


def kernel(x, edge_index, edge_attr, pos, batch_indices,
           W1_0, We_0, Ws_0, b_0,
           W1_1, We_1, Ws_1, b_1,
           W1_2, We_2, Ws_2, b_2):
    src = edge_index[0].astype(jnp.int32)
    dst = edge_index[1].astype(jnp.int32)
    bi2d = batch_indices.astype(jnp.int32).reshape(128, 128)

    rsrc, reid, rdl, cnts = _sc_prep(src, dst)
    z0, z1, z2 = _z_call(edge_attr, We_0, We_1, We_2,
                         b_0.reshape(1, D), b_1.reshape(1, D), b_2.reshape(1, D))
    y, sp = _pre_call(x, W1_0, Ws_0)
    agg = _sc_layer(y, z0, rsrc, reid, rdl, cnts)
    y, sp = _mid_call(agg, sp, W1_1, Ws_1)
    agg = _sc_layer(y, z1, rsrc, reid, rdl, cnts)
    y, sp = _mid_call(agg, sp, W1_2, Ws_2)
    agg = _sc_layer(y, z2, rsrc, reid, rdl, cnts)
    h3p = _fin_call(agg, sp)
    return _seq_call(bi2d, h3p)


# speedup vs baseline: 1.7529x; 1.0898x over previous
import functools

import jax
import jax.numpy as jnp
from jax import lax
from jax.experimental import pallas as pl
from jax.experimental.pallas import tpu as pltpu
from jax.experimental.pallas import tpu_sc as plsc

N = 16384
E = 262144
D = 256
DE = 16
B = 256
L = 128


_EB = 2048
_NB = 1024


def _pack_bf16(x):
    xb = x.astype(jnp.bfloat16).astype(jnp.float32)
    lo = jax.lax.bitcast_convert_type(xb[:, :D // 2], jnp.int32)
    hi = jax.lax.bitcast_convert_type(xb[:, D // 2:], jnp.int32)
    himask = jnp.int32(-65536)
    return jax.lax.shift_right_logical(lo, 16) | (hi & himask)


def _z_body(ea_ref, w0_ref, w1_ref, w2_ref, b0_ref, b1_ref, b2_ref,
            z0_ref, z1_ref, z2_ref):
    ea = ea_ref[...]
    z0_ref[...] = _pack_bf16(
        jnp.dot(ea, w0_ref[...], preferred_element_type=jnp.float32) + b0_ref[...])
    z1_ref[...] = _pack_bf16(
        jnp.dot(ea, w1_ref[...], preferred_element_type=jnp.float32) + b1_ref[...])
    z2_ref[...] = _pack_bf16(
        jnp.dot(ea, w2_ref[...], preferred_element_type=jnp.float32) + b2_ref[...])


def _z_call(ea, w0, w1, w2, b0, b1, b2):
    zspec = pl.BlockSpec((_EB, D // 2), lambda i: (i, 0))
    wspec = pl.BlockSpec((DE, D), lambda i: (0, 0))
    bspec = pl.BlockSpec((1, D), lambda i: (0, 0))
    return pl.pallas_call(
        _z_body,
        grid=(E // _EB,),
        in_specs=[pl.BlockSpec((_EB, DE), lambda i: (i, 0)),
                  wspec, wspec, wspec, bspec, bspec, bspec],
        out_specs=[zspec, zspec, zspec],
        out_shape=[jax.ShapeDtypeStruct((E, D // 2), jnp.int32)] * 3,
    )(ea, w0, w1, w2, b0, b1, b2)


def _pre_body(h_ref, w1_ref, ws_ref, y_ref, s_ref):
    h = h_ref[...]
    y_ref[...] = _pack_bf16(
        jnp.dot(h, w1_ref[...], preferred_element_type=jnp.float32))
    s_ref[...] = jnp.dot(h, ws_ref[...], preferred_element_type=jnp.float32)


def _mid_body(agg_ref, sp_ref, w1_ref, ws_ref, y_ref, s_ref):
    h = jnp.maximum(agg_ref[...] + sp_ref[...], 0.0)
    y_ref[...] = _pack_bf16(
        jnp.dot(h, w1_ref[...], preferred_element_type=jnp.float32))
    s_ref[...] = jnp.dot(h, ws_ref[...], preferred_element_type=jnp.float32)


def _h_specs():
    nspec = pl.BlockSpec((_NB, D), lambda i: (i, 0))
    yspec = pl.BlockSpec((_NB, D // 2), lambda i: (i, 0))
    wspec = pl.BlockSpec((D, D), lambda i: (0, 0))
    return nspec, yspec, wspec


def _pre_call(h, w1, ws):
    nspec, yspec, wspec = _h_specs()
    return pl.pallas_call(
        _pre_body,
        grid=(N // _NB,),
        in_specs=[nspec, wspec, wspec],
        out_specs=[yspec, nspec],
        out_shape=[jax.ShapeDtypeStruct((N, D // 2), jnp.int32),
                   jax.ShapeDtypeStruct((N, D), jnp.float32)],
    )(h, w1, ws)


def _mid_call(agg, sp, w1, ws):
    nspec, yspec, wspec = _h_specs()
    return pl.pallas_call(
        _mid_body,
        grid=(N // _NB,),
        in_specs=[nspec, nspec, wspec, wspec],
        out_specs=[yspec, nspec],
        out_shape=[jax.ShapeDtypeStruct((N, D // 2), jnp.int32),
                   jax.ShapeDtypeStruct((N, D), jnp.float32)],
    )(agg, sp, w1, ws)


_NP = N + 2 * L


def _fin_body(agg_ref, sp_ref, o_ref):
    i = pl.program_id(0)
    h = jnp.maximum(agg_ref[...] + sp_ref[...], 0.0)
    row = i * 128 + lax.broadcasted_iota(jnp.int32, (128, 1), 0)
    o_ref[...] = jnp.where(row < N, h, 0.0)


def _fin_call(agg, sp):
    nspec = pl.BlockSpec((128, D), lambda i: (jnp.minimum(i, 127), 0))
    return pl.pallas_call(
        _fin_body,
        grid=(_NP // 128,),
        in_specs=[nspec, nspec],
        out_specs=pl.BlockSpec((128, D), lambda i: (i, 0)),
        out_shape=jax.ShapeDtypeStruct((_NP, D), jnp.float32),
    )(agg, sp)


def _seq_body(bi_ref, h3_ref, o_ref):
    b = pl.program_id(0)
    bi = bi_ref[...]
    first = jnp.sum((bi < b).astype(jnp.int32))
    cnt = jnp.sum((bi == b).astype(jnp.int32))
    base = pl.multiple_of((first // 8) * 8, 8)
    rem = first - base
    window = h3_ref[pl.ds(base, L + 8), :]
    rows = pltpu.roll(window, (L + 8) - rem, 0)[:L]
    liota = lax.broadcasted_iota(jnp.int32, (L, 1), 0)
    o_ref[0] = jnp.where(liota < cnt, rows, 0.0)


def _seq_call(bi2d, h3p):
    return pl.pallas_call(
        _seq_body,
        grid=(B,),
        in_specs=[pl.BlockSpec((128, 128), lambda b: (0, 0)),
                  pl.BlockSpec((_NP, D), lambda b: (0, 0))],
        out_specs=pl.BlockSpec((1, L, D), lambda b: (b, 0, 0)),
        out_shape=jax.ShapeDtypeStruct((B, L, D), jnp.float32),
    )(bi2d, h3p)


_NW = 32
_OWN = N // 64
_S = 8192
_G = 32
_W = 256
_NBLK = E // _S
_CAP = E + 16 * _NBLK
_RC = 8192

_mesh = plsc.VectorSubcoreMesh(core_axis_name="c", subcore_axis_name="s")


def _prep_body(src_hbm, dst_hbm, rsrc_hbm, reid_hbm, rdl_hbm, cnts_hbm,
               dstb, srcb, c0s, c0e, c0d, c1s, c1e, c1d, cbuf, sem):
    c = lax.axis_index("c")
    s = lax.axis_index("s")
    w = s * 2 + c
    zero16i = jnp.zeros((16,), jnp.int32)
    iota16 = lax.iota(jnp.int32, 16)
    sent16 = jnp.full((16,), _OWN, jnp.int32)
    b0 = w
    b1 = _NW + w
    own0 = b0 * _OWN
    own1 = b1 * _OWN

    def zstage(i, _):
        c0s[pl.ds(i * 16, 16)] = zero16i
        c0e[pl.ds(i * 16, 16)] = zero16i
        c0d[pl.ds(i * 16, 16)] = zero16i
        c1s[pl.ds(i * 16, 16)] = zero16i
        c1e[pl.ds(i * 16, 16)] = zero16i
        c1d[pl.ds(i * 16, 16)] = zero16i
        return 0
    lax.fori_loop(0, (_S + _W + 16) // 16, zstage, 0)

    def block_body(bi, curs):
        cur0, cur1 = curs
        base = bi * _S
        pltpu.sync_copy(dst_hbm.at[pl.ds(base, _S)], dstb)
        pltpu.sync_copy(src_hbm.at[pl.ds(base, _S)], srcb)

        def scan(i, sts):
            st0, st1 = sts
            dv = dstb[pl.ds(i * 16, 16)]
            sv = srcb[pl.ds(i * 16, 16)]
            ev = iota16 + (base + i * 16)
            m0 = (dv >= own0) & (dv < own0 + _OWN)
            m1 = (dv >= own1) & (dv < own1 + _OWN)
            pos0 = st0 + plsc.cumsum(m0.astype(jnp.int32)) - 1
            pos1 = st1 + plsc.cumsum(m1.astype(jnp.int32)) - 1
            plsc.store_scatter(c0d, [pos0], dv - own0, mask=m0)
            plsc.store_scatter(c0s, [pos0], sv, mask=m0)
            plsc.store_scatter(c0e, [pos0], ev, mask=m0)
            plsc.store_scatter(c1d, [pos1], dv - own1, mask=m1)
            plsc.store_scatter(c1s, [pos1], sv, mask=m1)
            plsc.store_scatter(c1e, [pos1], ev, mask=m1)
            n0 = plsc.all_reduce_population_count(m0)
            n1 = plsc.all_reduce_population_count(m1)
            return st0 + n0[0], st1 + n1[0]

        n0, n1 = lax.fori_loop(0, _S // 16, scan, (0, 0))
        plsc.store_scatter(c0d, [n0 + iota16], sent16)
        plsc.store_scatter(c0s, [n0 + iota16], zero16i)
        plsc.store_scatter(c0e, [n0 + iota16], zero16i)
        plsc.store_scatter(c1d, [n1 + iota16], sent16)
        plsc.store_scatter(c1s, [n1 + iota16], zero16i)
        plsc.store_scatter(c1e, [n1 + iota16], zero16i)
        np0 = ((n0 + 15) // 16) * 16
        np1 = ((n1 + 15) // 16) * 16

        def flush0(k, _):
            o = pl.multiple_of(b0 * _CAP + cur0 + k * _W, 16)
            pltpu.sync_copy(c0s.at[pl.ds(k * _W, _W)], rsrc_hbm.at[pl.ds(o, _W)])
            pltpu.sync_copy(c0e.at[pl.ds(k * _W, _W)], reid_hbm.at[pl.ds(o, _W)])
            pltpu.sync_copy(c0d.at[pl.ds(k * _W, _W)], rdl_hbm.at[pl.ds(o, _W)])
            return 0

        def flush1(k, _):
            o = pl.multiple_of(b1 * _CAP + cur1 + k * _W, 16)
            pltpu.sync_copy(c1s.at[pl.ds(k * _W, _W)], rsrc_hbm.at[pl.ds(o, _W)])
            pltpu.sync_copy(c1e.at[pl.ds(k * _W, _W)], reid_hbm.at[pl.ds(o, _W)])
            pltpu.sync_copy(c1d.at[pl.ds(k * _W, _W)], rdl_hbm.at[pl.ds(o, _W)])
            return 0

        lax.fori_loop(0, (np0 + _W - 1) // _W, flush0, 0)
        lax.fori_loop(0, (np1 + _W - 1) // _W, flush1, 0)
        return cur0 + np0, cur1 + np1

    t0, t1 = lax.fori_loop(0, _NBLK, block_body, (0, 0))
    cbuf[pl.ds(0, 16)] = jnp.full((16,), t0, jnp.int32)
    pltpu.sync_copy(cbuf, cnts_hbm.at[pl.ds(b0 * 16, 16)])
    cbuf[pl.ds(0, 16)] = jnp.full((16,), t1, jnp.int32)
    pltpu.sync_copy(cbuf, cnts_hbm.at[pl.ds(b1 * 16, 16)])


_sc_prep = functools.partial(
    pl.kernel,
    mesh=_mesh,
    compiler_params=pltpu.CompilerParams(needs_layout_passes=False),
    out_type=[
        jax.ShapeDtypeStruct((64 * _CAP,), jnp.int32),
        jax.ShapeDtypeStruct((64 * _CAP,), jnp.int32),
        jax.ShapeDtypeStruct((64 * _CAP,), jnp.int32),
        jax.ShapeDtypeStruct((64 * 16,), jnp.int32),
    ],
    scratch_types=[
        pltpu.VMEM((_S,), jnp.int32),
        pltpu.VMEM((_S,), jnp.int32),
        pltpu.VMEM((_S + _W + 16,), jnp.int32),
        pltpu.VMEM((_S + _W + 16,), jnp.int32),
        pltpu.VMEM((_S + _W + 16,), jnp.int32),
        pltpu.VMEM((_S + _W + 16,), jnp.int32),
        pltpu.VMEM((_S + _W + 16,), jnp.int32),
        pltpu.VMEM((_S + _W + 16,), jnp.int32),
        pltpu.VMEM((16,), jnp.int32),
        pltpu.SemaphoreType.DMA,
    ],
)(_prep_body)


def _layer_body(y_hbm, z_hbm, rsrc_hbm, reid_hbm, rdl_hbm, cnts_hbm, agg_hbm,
                rc_s, rc_e, rc_d, cbuf,
                w0, w1, w2, w3, v0, v1, v2, v3, acc,
                sz0, sz1, sz2, sz3, sy0, sy1, sy2, sy3):
    c = lax.axis_index("c")
    s = lax.axis_index("s")
    w = s * 2 + c
    zero16f = jnp.zeros((16,), jnp.float32)
    zero16i = jnp.zeros((16,), jnp.int32)
    iota16 = lax.iota(jnp.int32, 16)
    wbuf = (w0, w1, w2, w3)
    ybuf = (v0, v1, v2, v3)
    semz = (sz0, sz1, sz2, sz3)
    semy = (sy0, sy1, sy2, sy3)
    himask = jnp.full((16,), -65536, jnp.int32)

    def zrc(i, _):
        rc_s[pl.ds(i * 16, 16)] = zero16i
        rc_e[pl.ds(i * 16, 16)] = zero16i
        rc_d[pl.ds(i * 16, 16)] = zero16i
        return 0
    lax.fori_loop(0, _RC // 16, zrc, 0)

    def zero_acc(i, _):
        for j in range(D // 16):
            acc[i, pl.ds(j * 16, 16)] = zero16f
        return 0

    for p in range(2):
        bucket = p * _NW + w
        own_base = bucket * _OWN
        rbase = bucket * _CAP
        pltpu.sync_copy(cnts_hbm.at[pl.ds(bucket * 16, 16)], cbuf)
        cnt = cbuf[pl.ds(0, 16)][0]
        lax.fori_loop(0, _OWN + 1, zero_acc, 0)

        nchunk = (cnt + _RC - 1) // _RC

        def chunk_body(ci, _):
            c0 = ci * _RC
            n_in = jnp.minimum(_RC, cnt - c0)
            f0 = pl.multiple_of(rbase + c0, 16)
            pltpu.sync_copy(rsrc_hbm.at[pl.ds(f0, _RC)], rc_s)
            pltpu.sync_copy(reid_hbm.at[pl.ds(f0, _RC)], rc_e)
            pltpu.sync_copy(rdl_hbm.at[pl.ds(f0, _RC)], rc_d)
            ngrp = (n_in + _G - 1) // _G

            def issue(u, g):
                pltpu.async_copy(
                    z_hbm.at[rc_e.at[pl.ds(g * _G, _G)]], wbuf[u], semz[u])
                pltpu.async_copy(
                    y_hbm.at[rc_s.at[pl.ds(g * _G, _G)]], ybuf[u], semy[u])

            def drain(u):
                pltpu.make_async_copy(
                    z_hbm.at[pl.ds(0, _G)], wbuf[u], semz[u]).wait()
                pltpu.make_async_copy(
                    y_hbm.at[pl.ds(0, _G)], ybuf[u], semy[u]).wait()

            def compute(goff, wrow, yrow):
                nrows = jnp.minimum(_G, n_in - goff)

                def rowfn(i2, _):
                    i = i2 * 2
                    dvis = [plsc.load_gather(
                        rc_d, [jnp.full((16,), goff + i + u, jnp.int32)])
                        for u in range(2)]
                    for j in range(D // 32):
                        cola = j * 16 + iota16
                        colb = (D // 2) + j * 16 + iota16
                        for u in range(2):
                            zw = wrow[i + u, pl.ds(j * 16, 16)]
                            yw = yrow[i + u, pl.ds(j * 16, 16)]
                            za = plsc.bitcast(zw << 16, jnp.float32)
                            ya = plsc.bitcast(yw << 16, jnp.float32)
                            zb = plsc.bitcast(zw & himask, jnp.float32)
                            yb = plsc.bitcast(yw & himask, jnp.float32)
                            plsc.addupdate_scatter(
                                acc, [dvis[u], cola],
                                jnp.maximum(za + ya, 0.0))
                            plsc.addupdate_scatter(
                                acc, [dvis[u], colb],
                                jnp.maximum(zb + yb, 0.0))
                    return 0

                lax.fori_loop(0, nrows // 2, rowfn, 0)

            for u in range(4):
                @pl.when(u < ngrp)
                def _(u=u):
                    issue(u, u)

            def quad(q, _):
                g0 = q * 4
                for u in range(4):
                    @pl.when(g0 + u < ngrp)
                    def _(u=u):
                        drain(u)
                        compute((g0 + u) * _G, wbuf[u], ybuf[u])

                        @pl.when(g0 + u + 4 < ngrp)
                        def _():
                            issue(u, g0 + u + 4)
                return 0

            lax.fori_loop(0, (ngrp + 3) // 4, quad, 0)
            return 0

        lax.fori_loop(0, nchunk, chunk_body, 0)
        pltpu.sync_copy(acc.at[pl.ds(0, _OWN)],
                        agg_hbm.at[pl.ds(own_base, _OWN)])


_sc_layer = functools.partial(
    pl.kernel,
    mesh=_mesh,
    compiler_params=pltpu.CompilerParams(needs_layout_passes=False),
    out_type=jax.ShapeDtypeStruct((N, D), jnp.float32),
    scratch_types=[
        pltpu.VMEM((_RC,), jnp.int32),
        pltpu.VMEM((_RC,), jnp.int32),
        pltpu.VMEM((_RC,), jnp.int32),
        pltpu.VMEM((16,), jnp.int32),
        pltpu.VMEM((_G, D // 2), jnp.int32),
        pltpu.VMEM((_G, D // 2), jnp.int32),
        pltpu.VMEM((_G, D // 2), jnp.int32),
        pltpu.VMEM((_G, D // 2), jnp.int32),
        pltpu.VMEM((_G, D // 2), jnp.int32),
        pltpu.VMEM((_G, D // 2), jnp.int32),
        pltpu.VMEM((_G, D // 2), jnp.int32),
        pltpu.VMEM((_G, D // 2), jnp.int32),
        pltpu.VMEM((_OWN + 1, D), jnp.float32),
        pltpu.SemaphoreType.DMA,
        pltpu.SemaphoreType.DMA,
        pltpu.SemaphoreType.DMA,
        pltpu.SemaphoreType.DMA,
        pltpu.SemaphoreType.DMA,
        pltpu.SemaphoreType.DMA,
        pltpu.SemaphoreType.DMA,
        pltpu.SemaphoreType.DMA,
    ],
)(_layer_body)


def kernel(x, edge_index, edge_attr, pos, batch_indices,
           W1_0, We_0, Ws_0, b_0,
           W1_1, We_1, Ws_1, b_1,
           W1_2, We_2, Ws_2, b_2):
    src = edge_index[0].astype(jnp.int32)
    dst = edge_index[1].astype(jnp.int32)
    bi2d = batch_indices.astype(jnp.int32).reshape(128, 128)

    rsrc, reid, rdl, cnts = _sc_prep(src, dst)
    z0, z1, z2 = _z_call(edge_attr, We_0, We_1, We_2,
                         b_0.reshape(1, D), b_1.reshape(1, D), b_2.reshape(1, D))
    y, sp = _pre_call(x, W1_0, Ws_0)
    agg = _sc_layer(y, z0, rsrc, reid, rdl, cnts)
    y, sp = _mid_call(agg, sp, W1_1, Ws_1)
    agg = _sc_layer(y, z1, rsrc, reid, rdl, cnts)
    y, sp = _mid_call(agg, sp, W1_2, Ws_2)
    agg = _sc_layer(y, z2, rsrc, reid, rdl, cnts)
    h3p = _fin_call(agg, sp)
    return _seq_call(bi2d, h3p)
